# Initial kernel scaffold; baseline (speedup 1.0000x reference)
import jax
import jax.numpy as jnp
from jax.experimental import pallas as pl


def kernel(x, w_reduce, b_reduce, w_expand, b_expand):
    raise NotImplementedError("write your pallas kernel here")


# speedup vs baseline: 1.7600x; 1.7600x over previous
import functools

import jax
import jax.numpy as jnp
from jax.experimental import pallas as pl
from jax.experimental.pallas import tpu as pltpu


def _se_fused_kernel(x_ref, w1_ref, b1_ref, w2_ref, b2_ref, o_ref, *, inv_hw):
    x = x_ref[0].astype(jnp.float32)
    pooled = jnp.sum(x, axis=-1, keepdims=True) * inv_hw
    h = jnp.dot(w1_ref[...], pooled,
                preferred_element_type=jnp.float32)
    h = jnp.maximum(h + b1_ref[...], 0.0)
    z = jnp.dot(w2_ref[...], h,
                preferred_element_type=jnp.float32)
    g = jax.nn.sigmoid(z + b2_ref[...])
    o_ref[0] = (x * g).astype(o_ref.dtype)


def kernel(x, w_reduce, b_reduce, w_expand, b_expand):
    N, C, H, W = x.shape
    hw = H * W
    cr = w_reduce.shape[0]

    xf = x.reshape(N, C, hw)
    w1 = w_reduce.astype(jnp.float32)
    b1 = b_reduce.astype(jnp.float32)
    w2 = w_expand.astype(jnp.float32)
    b2 = b_expand.astype(jnp.float32)

    y = pl.pallas_call(
        functools.partial(_se_fused_kernel, inv_hw=1.0 / float(hw)),
        out_shape=jax.ShapeDtypeStruct((N, C, hw), x.dtype),
        grid=(N,),
        in_specs=[
            pl.BlockSpec((1, C, hw), lambda n: (n, 0, 0)),
            pl.BlockSpec((cr, C), lambda n: (0, 0)),
            pl.BlockSpec((cr, 1), lambda n: (0, 0)),
            pl.BlockSpec((C, cr), lambda n: (0, 0)),
            pl.BlockSpec((C, 1), lambda n: (0, 0)),
        ],
        out_specs=pl.BlockSpec((1, C, hw), lambda n: (n, 0, 0)),
        compiler_params=pltpu.CompilerParams(
            dimension_semantics=("parallel",)),
        cost_estimate=pl.CostEstimate(
            flops=int(N * C * hw + 4 * N * C * cr + 2 * N * C * hw),
            transcendentals=int(N * C),
            bytes_accessed=int(2 * xf.size * x.dtype.itemsize
                               + (w1.size + b1.size + w2.size + b2.size) * 4),
        ),
    )(xf, w1, b1, w2, b2)

    return y.reshape(N, C, H, W)
